# baseline (speedup 1.0000x reference)
import jax
import jax.numpy as jnp
from jax.experimental import pallas as pl


def kernel(x, edge_index, edge_attr, opt_edge, em_W, em_b, emo_W, emo_b,
           le1_W, le1_b, nn1_W, nn1_b, le2_W, le2_b, nn2_W, nn2_b,
           le3_W, le3_b, nn3_W, nn3_b, dec_W, dec_b):
    N = x.shape[0]
    a = edge_attr[:, 0]
    o = opt_edge[:, 0]
    src = edge_index[0].astype(jnp.int32)
    dst = edge_index[1].astype(jnp.int32)
    u = jnp.concatenate([em_W[0], jnp.zeros_like(emo_W[0])])
    v = jnp.concatenate([jnp.zeros_like(em_W[0]), emo_W[0]])
    c = jnp.concatenate([em_b, emo_b])
    p1 = (u @ le1_W)[0]; q1 = (v @ le1_W)[0]; r1 = (c @ le1_W + le1_b)[0]
    p2 = u @ le2_W; q2 = v @ le2_W; r2 = c @ le2_W + le2_b
    p3 = u @ le3_W; q3 = v @ le3_W; r3 = c @ le3_W + le3_b
    w1 = nn1_W[0]; b1 = nn1_b
    d2 = b1 + r2

    xs = x[:, 0]
    m1 = jax.nn.relu(xs[src] + a * p1 + o * q1 + r1)
    aggr1 = jax.ops.segment_sum(m1, dst, num_segments=N)
    s1 = aggr1 + xs
    msg2 = jax.nn.relu(s1[src][:, None] * w1[None, :]
                       + a[:, None] * p2[None, :]
                       + o[:, None] * q2[None, :] + d2[None, :])
    aggr2 = jax.ops.segment_sum(msg2, dst, num_segments=N)
    out2 = aggr2 + s1[:, None] * w1[None, :] + b1[None, :]
    h2 = out2 @ nn2_W + nn2_b
    msg3 = jax.nn.relu(h2[src] + a[:, None] * p3[None, :]
                       + o[:, None] * q3[None, :] + r3[None, :])
    aggr3 = jax.ops.segment_sum(msg3, dst, num_segments=N)
    out3 = aggr3 + h2
    h3 = out3 @ nn3_W + nn3_b
    g1 = h3 @ dec_W[:128] + dec_b
    g2 = h3 @ dec_W[128:]
    return g1[src] + g2[dst]


# speedup vs baseline: 6.6507x; 6.6507x over previous
import functools

import jax
import jax.numpy as jnp
from jax import lax
from jax.experimental import pallas as pl
from jax.experimental.pallas import tpu as pltpu
from jax.experimental.pallas import tpu_sc as plsc

N = 50000
E = 800000
H = 128
NC = 2
NS = 16

NPAD = 50176
EPAD = 802816
EPT32 = EPAD // 32
EPT16 = EPAD // 16
ECH = 512
NSL = NPAD // 16
CH = NPAD // 2
CHT = CH // 16
ACCR = CH + 16

_vmesh = plsc.VectorSubcoreMesh(
    core_axis_name="c", subcore_axis_name="s", num_cores=NC, num_subcores=NS)
_sc_params = pltpu.CompilerParams(
    needs_layout_passes=False, use_tc_tiling_on_sc=False)


@functools.partial(
    pl.kernel,
    out_type=(jax.ShapeDtypeStruct((NPAD,), jnp.float32),
              jax.ShapeDtypeStruct((EPAD,), jnp.float32)),
    mesh=_vmesh,
    compiler_params=_sc_params,
    scratch_types=[
        pltpu.VMEM((NPAD,), jnp.float32),
        pltpu.VMEM((NPAD,), jnp.float32),
        pltpu.VMEM((ECH,), jnp.int32),
        pltpu.VMEM((ECH,), jnp.int32),
        pltpu.VMEM((ECH,), jnp.float32),
        pltpu.VMEM((ECH,), jnp.float32),
        pltpu.VMEM((16,), jnp.float32),
        pltpu.VMEM_SHARED((NS * NSL,), jnp.float32),
        pltpu.VMEM((NSL,), jnp.float32),
        pltpu.VMEM((NSL,), jnp.float32),
        pltpu.VMEM_SHARED((NPAD,), jnp.float32),
        pltpu.VMEM((ECH,), jnp.float32),
    ],
)
def _k1(xs_hbm, src_hbm, dst_hbm, a_hbm, o_hbm, c1_hbm,
        s1_hbm, s1src_hbm,
        x_tab, acc, srcb, dstb, ab, ob, c1v, stage, rbuf, sacc, s1sh, outb):
    c = lax.axis_index("c")
    s = lax.axis_index("s")
    pltpu.sync_copy(xs_hbm, x_tab)
    pltpu.sync_copy(c1_hbm, c1v)
    cv = c1v[...]
    p1 = cv[0]
    q1 = cv[1]
    r1 = cv[2]

    @pl.loop(0, NPAD, step=16)
    def _zero(i):
        acc[pl.ds(i, 16)] = jnp.zeros((16,), jnp.float32)

    base = s * EPT16

    @pl.loop(0, EPT16, step=ECH)
    def _chunk(i):
        pltpu.sync_copy(src_hbm.at[pl.ds(base + i, ECH)], srcb)
        pltpu.sync_copy(dst_hbm.at[pl.ds(base + i, ECH)], dstb)
        pltpu.sync_copy(a_hbm.at[pl.ds(base + i, ECH)], ab)
        pltpu.sync_copy(o_hbm.at[pl.ds(base + i, ECH)], ob)

        @pl.loop(0, ECH, step=16)
        def _grp(j):
            sv = srcb[pl.ds(j, 16)]
            dv = dstb[pl.ds(j, 16)]
            av = ab[pl.ds(j, 16)]
            ov = ob[pl.ds(j, 16)]
            xg = plsc.load_gather(x_tab, [sv])
            m = jnp.maximum(xg + av * p1 + ov * q1 + r1, 0.0)
            plsc.addupdate_scatter(acc, [dv], m)

    @pl.loop(0, NSL, step=16)
    def _zs(i):
        sacc[pl.ds(i, 16)] = jnp.zeros((16,), jnp.float32)

    @pl.loop(0, NS)
    def _round(r):
        k = (s + r) & (NS - 1)
        pltpu.sync_copy(acc.at[pl.ds(k * NSL, NSL)], stage.at[pl.ds(k * NSL, NSL)])
        plsc.subcore_barrier()
        pltpu.sync_copy(stage.at[pl.ds(s * NSL, NSL)], rbuf)

        @pl.loop(0, NSL, step=16)
        def _add(i):
            sacc[pl.ds(i, 16)] = sacc[pl.ds(i, 16)] + rbuf[pl.ds(i, 16)]

        plsc.subcore_barrier()

    @pl.loop(0, NSL, step=16)
    def _s1(i):
        sacc[pl.ds(i, 16)] = sacc[pl.ds(i, 16)] + x_tab[pl.ds(s * NSL + i, 16)]

    pltpu.sync_copy(sacc, s1sh.at[pl.ds(s * NSL, NSL)])

    @pl.when(c == 0)
    def _():
        pltpu.sync_copy(sacc, s1_hbm.at[pl.ds(s * NSL, NSL)])

    plsc.subcore_barrier()
    pltpu.sync_copy(s1sh, x_tab)

    base2 = (c * NS + s) * EPT32

    @pl.loop(0, EPT32, step=ECH)
    def _gch(i):
        pltpu.sync_copy(src_hbm.at[pl.ds(base2 + i, ECH)], srcb)

        @pl.loop(0, ECH, step=16)
        def _g(j):
            sv = srcb[pl.ds(j, 16)]
            outb[pl.ds(j, 16)] = plsc.load_gather(x_tab, [sv])

        pltpu.sync_copy(outb, s1src_hbm.at[pl.ds(base2 + i, ECH)])


@functools.partial(
    pl.kernel,
    out_type=(jax.ShapeDtypeStruct((NPAD, 64), jnp.float32),
              jax.ShapeDtypeStruct((NPAD, 64), jnp.float32)),
    mesh=_vmesh,
    compiler_params=_sc_params,
    scratch_types=[
        pltpu.VMEM((ECH,), jnp.int32),
        pltpu.VMEM((ECH,), jnp.float32),
        pltpu.VMEM((ECH,), jnp.float32),
        pltpu.VMEM((ECH,), jnp.float32),
        pltpu.VMEM((1024,), jnp.float32),
        pltpu.VMEM((128, 64), jnp.float32),
        pltpu.VMEM((128,), jnp.int32),
        pltpu.VMEM_SHARED((ACCR, 64), jnp.float32),
    ],
)
def _k2(t_hbm, dst_hbm, a_hbm, o_hbm, cf_hbm, z_hbm, outa_hbm, outb_hbm,
        dstb, tb, ab, ob, cf, msgb, idxb, accs):
    c = lax.axis_index("c")
    s = lax.axis_index("s")
    pltpu.sync_copy(cf_hbm, cf)
    lane0 = c * 64
    w1q = [cf[pl.ds(0 * 128 + lane0 + q * 16, 16)] for q in range(4)]
    p2q = [cf[pl.ds(1 * 128 + lane0 + q * 16, 16)] for q in range(4)]
    q2q = [cf[pl.ds(2 * 128 + lane0 + q * 16, 16)] for q in range(4)]
    d2q = [cf[pl.ds(3 * 128 + lane0 + q * 16, 16)] for q in range(4)]

    for p in range(2):
        pltpu.sync_copy(z_hbm.at[pl.ds(s * CHT, CHT)], accs.at[pl.ds(s * CHT, CHT)])
        plsc.subcore_barrier()
        base = s * EPT16

        @pl.loop(0, EPT16, step=ECH)
        def _chunk(i):
            pltpu.sync_copy(dst_hbm.at[pl.ds(base + i, ECH)], dstb)
            pltpu.sync_copy(t_hbm.at[pl.ds(base + i, ECH)], tb)
            pltpu.sync_copy(a_hbm.at[pl.ds(base + i, ECH)], ab)
            pltpu.sync_copy(o_hbm.at[pl.ds(base + i, ECH)], ob)

            for sub in range(4):
                @pl.loop(0, 8)
                def _grp(g):
                    e0 = sub * 128 + g * 16
                    dv = dstb[pl.ds(e0, 16)]
                    loc = dv - p * CH
                    inr = (loc >= 0) & (loc < CH)
                    idxb[pl.ds(g * 16, 16)] = jnp.where(inr, loc, CH)
                    tv = tb[pl.ds(e0, 16)]
                    av = ab[pl.ds(e0, 16)]
                    ov = ob[pl.ds(e0, 16)]
                    for l in range(16):
                        t_ = tv[l]
                        a_ = av[l]
                        o_ = ov[l]
                        row = g * 16 + l
                        for q in range(4):
                            msgb[row, pl.ds(q * 16, 16)] = jnp.maximum(
                                t_ * w1q[q] + a_ * p2q[q] + o_ * q2q[q] + d2q[q], 0.0)

                pltpu.sync_copy(msgb, accs.at[idxb], add=True)

        plsc.subcore_barrier()

        @pl.when(c == 0)
        def _():
            pltpu.sync_copy(accs.at[pl.ds(s * CHT, CHT)],
                            outa_hbm.at[pl.ds(p * CH + s * CHT, CHT)])

        @pl.when(c == 1)
        def _():
            pltpu.sync_copy(accs.at[pl.ds(s * CHT, CHT)],
                            outb_hbm.at[pl.ds(p * CH + s * CHT, CHT)])

        plsc.subcore_barrier()


@functools.partial(
    pl.kernel,
    out_type=(jax.ShapeDtypeStruct((NPAD, 64), jnp.float32),
              jax.ShapeDtypeStruct((NPAD, 64), jnp.float32)),
    mesh=_vmesh,
    compiler_params=_sc_params,
    scratch_types=[
        pltpu.VMEM((ECH,), jnp.int32),
        pltpu.VMEM((ECH,), jnp.int32),
        pltpu.VMEM((ECH,), jnp.float32),
        pltpu.VMEM((ECH,), jnp.float32),
        pltpu.VMEM((1024,), jnp.float32),
        pltpu.VMEM((128,), jnp.int32),
        pltpu.VMEM((128, 64), jnp.float32),
        pltpu.VMEM((128,), jnp.int32),
        pltpu.VMEM_SHARED((ACCR, 64), jnp.float32),
    ],
)
def _k4(h2_hbm, src_hbm, dst_hbm, a_hbm, o_hbm, cf_hbm, z_hbm,
        outa_hbm, outb_hbm,
        srcb, dstb, ab, ob, cf, gidx, grows, idxb, accs):
    c = lax.axis_index("c")
    s = lax.axis_index("s")
    pltpu.sync_copy(cf_hbm, cf)
    lane0 = c * 64
    p3q = [cf[pl.ds(4 * 128 + lane0 + q * 16, 16)] for q in range(4)]
    q3q = [cf[pl.ds(5 * 128 + lane0 + q * 16, 16)] for q in range(4)]
    r3q = [cf[pl.ds(6 * 128 + lane0 + q * 16, 16)] for q in range(4)]
    tab0 = c * NPAD

    for p in range(2):
        pltpu.sync_copy(z_hbm.at[pl.ds(s * CHT, CHT)], accs.at[pl.ds(s * CHT, CHT)])
        plsc.subcore_barrier()
        base = s * EPT16

        @pl.loop(0, EPT16, step=ECH)
        def _chunk(i):
            pltpu.sync_copy(src_hbm.at[pl.ds(base + i, ECH)], srcb)
            pltpu.sync_copy(dst_hbm.at[pl.ds(base + i, ECH)], dstb)
            pltpu.sync_copy(a_hbm.at[pl.ds(base + i, ECH)], ab)
            pltpu.sync_copy(o_hbm.at[pl.ds(base + i, ECH)], ob)

            for sub in range(4):
                @pl.loop(0, 8)
                def _gi(g):
                    e0 = sub * 128 + g * 16
                    gidx[pl.ds(g * 16, 16)] = srcb[pl.ds(e0, 16)] + tab0
                    dv = dstb[pl.ds(e0, 16)]
                    loc = dv - p * CH
                    inr = (loc >= 0) & (loc < CH)
                    idxb[pl.ds(g * 16, 16)] = jnp.where(inr, loc, CH)

                pltpu.sync_copy(h2_hbm.at[gidx], grows)

                @pl.loop(0, 8)
                def _grp(g):
                    e0 = sub * 128 + g * 16
                    av = ab[pl.ds(e0, 16)]
                    ov = ob[pl.ds(e0, 16)]
                    for l in range(16):
                        a_ = av[l]
                        o_ = ov[l]
                        row = g * 16 + l
                        for q in range(4):
                            v = grows[row, pl.ds(q * 16, 16)]
                            grows[row, pl.ds(q * 16, 16)] = jnp.maximum(
                                v + a_ * p3q[q] + o_ * q3q[q] + r3q[q], 0.0)

                pltpu.sync_copy(grows, accs.at[idxb], add=True)

        plsc.subcore_barrier()

        @pl.when(c == 0)
        def _():
            pltpu.sync_copy(accs.at[pl.ds(s * CHT, CHT)],
                            outa_hbm.at[pl.ds(p * CH + s * CHT, CHT)])

        @pl.when(c == 1)
        def _():
            pltpu.sync_copy(accs.at[pl.ds(s * CHT, CHT)],
                            outb_hbm.at[pl.ds(p * CH + s * CHT, CHT)])

        plsc.subcore_barrier()


@functools.partial(
    pl.kernel,
    out_type=jax.ShapeDtypeStruct((EPAD,), jnp.float32),
    mesh=_vmesh,
    compiler_params=_sc_params,
    scratch_types=[
        pltpu.VMEM((NPAD,), jnp.float32),
        pltpu.VMEM((NPAD,), jnp.float32),
        pltpu.VMEM((ECH,), jnp.int32),
        pltpu.VMEM((ECH,), jnp.int32),
        pltpu.VMEM((ECH,), jnp.float32),
    ],
)
def _k6(g1_hbm, g2_hbm, src_hbm, dst_hbm, res_hbm,
        g1t, g2t, srcb, dstb, outb):
    c = lax.axis_index("c")
    s = lax.axis_index("s")
    pltpu.sync_copy(g1_hbm, g1t)
    pltpu.sync_copy(g2_hbm, g2t)
    base = (c * NS + s) * EPT32

    @pl.loop(0, EPT32, step=ECH)
    def _chunk(i):
        pltpu.sync_copy(src_hbm.at[pl.ds(base + i, ECH)], srcb)
        pltpu.sync_copy(dst_hbm.at[pl.ds(base + i, ECH)], dstb)

        @pl.loop(0, ECH, step=16)
        def _g(j):
            sv = srcb[pl.ds(j, 16)]
            dv = dstb[pl.ds(j, 16)]
            outb[pl.ds(j, 16)] = (plsc.load_gather(g1t, [sv])
                                  + plsc.load_gather(g2t, [dv]))

        pltpu.sync_copy(outb, res_hbm.at[pl.ds(base + i, ECH)])


_BN = 1024


def _k3_body(a_ref, b_ref, s1_ref, wa_ref, wb_ref, tu_ref, out_ref):
    hb = (jnp.dot(a_ref[...], wa_ref[0], preferred_element_type=jnp.float32)
          + jnp.dot(b_ref[...], wb_ref[0], preferred_element_type=jnp.float32)
          + s1_ref[...] * tu_ref[0, 0:1, :] + tu_ref[0, 1:2, :])
    out_ref[0] = hb


def _k3(aggr_a, aggr_b, s1c, w2a, w2b, tu):
    return pl.pallas_call(
        _k3_body,
        grid=(2, NPAD // _BN),
        in_specs=[
            pl.BlockSpec((_BN, 64), lambda h, i: (i, 0)),
            pl.BlockSpec((_BN, 64), lambda h, i: (i, 0)),
            pl.BlockSpec((_BN, 1), lambda h, i: (i, 0)),
            pl.BlockSpec((1, 64, 64), lambda h, i: (h, 0, 0)),
            pl.BlockSpec((1, 64, 64), lambda h, i: (h, 0, 0)),
            pl.BlockSpec((1, 2, 64), lambda h, i: (h, 0, 0)),
        ],
        out_specs=pl.BlockSpec((1, _BN, 64), lambda h, i: (h, i, 0)),
        out_shape=jax.ShapeDtypeStruct((2, NPAD, 64), jnp.float32),
    )(aggr_a, aggr_b, s1c, w2a, w2b, tu)


def _k5_body(a3_ref, b3_ref, h2a_ref, h2b_ref, ga_ref, gb_ref, gb2_ref, out_ref):
    xa = a3_ref[...] + h2a_ref[...]
    xb = b3_ref[...] + h2b_ref[...]
    out_ref[...] = (jnp.dot(xa, ga_ref[...], preferred_element_type=jnp.float32)
                    + jnp.dot(xb, gb_ref[...], preferred_element_type=jnp.float32)
                    + gb2_ref[...])


def _k5(a3, b3, h2a, h2b, ga, gb, gbias):
    return pl.pallas_call(
        _k5_body,
        grid=(NPAD // _BN,),
        in_specs=[
            pl.BlockSpec((_BN, 64), lambda i: (i, 0)),
            pl.BlockSpec((_BN, 64), lambda i: (i, 0)),
            pl.BlockSpec((_BN, 64), lambda i: (i, 0)),
            pl.BlockSpec((_BN, 64), lambda i: (i, 0)),
            pl.BlockSpec((64, 8), lambda i: (0, 0)),
            pl.BlockSpec((64, 8), lambda i: (0, 0)),
            pl.BlockSpec((1, 8), lambda i: (0, 0)),
        ],
        out_specs=pl.BlockSpec((_BN, 8), lambda i: (i, 0)),
        out_shape=jax.ShapeDtypeStruct((NPAD, 8), jnp.float32),
    )(a3, b3, h2a, h2b, ga, gb, gbias)


def kernel(x, edge_index, edge_attr, opt_edge, em_W, em_b, emo_W, emo_b,
           le1_W, le1_b, nn1_W, nn1_b, le2_W, le2_b, nn2_W, nn2_b,
           le3_W, le3_b, nn3_W, nn3_b, dec_W, dec_b):
    a = edge_attr[:, 0]
    o = opt_edge[:, 0]
    src = edge_index[0].astype(jnp.int32)
    dst = edge_index[1].astype(jnp.int32)

    u = jnp.concatenate([em_W[0], jnp.zeros_like(emo_W[0])])
    v = jnp.concatenate([jnp.zeros_like(em_W[0]), emo_W[0]])
    cc = jnp.concatenate([em_b, emo_b])
    p1 = (u @ le1_W)[0]; q1 = (v @ le1_W)[0]; r1 = (cc @ le1_W + le1_b)[0]
    p2 = u @ le2_W; q2 = v @ le2_W; r2 = cc @ le2_W + le2_b
    p3 = u @ le3_W; q3 = v @ le3_W; r3 = cc @ le3_W + le3_b
    w1 = nn1_W[0]; b1 = nn1_b
    d2 = b1 + r2

    xs = jnp.pad(x[:, 0], (0, NPAD - N))
    srcp = jnp.pad(src, (0, EPAD - E), constant_values=N)
    dstp = jnp.pad(dst, (0, EPAD - E), constant_values=N)
    ap = jnp.pad(a, (0, EPAD - E))
    op = jnp.pad(o, (0, EPAD - E))
    c1 = jnp.zeros((16,), jnp.float32).at[0].set(p1).at[1].set(q1).at[2].set(r1)
    cf = jnp.stack([w1, p2, q2, d2, p3, q3, r3,
                    jnp.zeros_like(w1)]).reshape(-1)
    zrows = jnp.zeros((CH, 64), jnp.float32)

    s1p, t_edge = _k1(xs, srcp, dstp, ap, op, c1)

    aggr2a, aggr2b = _k2(t_edge, dstp, ap, op, cf, zrows)

    t2 = w1 @ nn2_W
    u2 = b1 @ nn2_W + nn2_b
    tu = jnp.stack([t2, u2]).reshape(2, 2, 64).transpose(1, 0, 2)
    w2a = nn2_W[:64].reshape(64, 2, 64).transpose(1, 0, 2)
    w2b = nn2_W[64:].reshape(64, 2, 64).transpose(1, 0, 2)
    h2 = _k3(aggr2a, aggr2b, s1p[:, None], w2a, w2b, tu)
    h2cat = h2.reshape(2 * NPAD, 64)

    aggr3a, aggr3b = _k4(h2cat, srcp, dstp, ap, op, cf, zrows)

    G = jnp.zeros((H, 8), jnp.float32)
    G = G.at[:, 0].set(dec_W[:128, 0]).at[:, 1].set(dec_W[128:, 0])
    W3G = nn3_W @ G
    gbias = (nn3_b @ G + jnp.zeros((8,), jnp.float32).at[0].set(dec_b[0]))[None, :]
    gout = _k5(aggr3a, aggr3b, h2[0], h2[1], W3G[:64], W3G[64:], gbias)

    res = _k6(gout[:, 0], gout[:, 1], srcp, dstp)
    return res[:E][:, None]
